# baseline (speedup 1.0000x reference)
import functools

import jax
import jax.numpy as jnp
from jax.experimental import pallas as pl
from jax.experimental.pallas import tpu as pltpu

NTOK = 4096
DIMIN = 2048
WIDTH = 16384
KVAL = 64


TB1 = 512
WB1 = 2048


def _encode_body(x_ref, ae_ref, hr_ref, h16_ref):
    acc = jax.lax.dot_general(
        x_ref[...], ae_ref[...], (((1,), (1,)), ((), ())),
        preferred_element_type=jnp.float32)
    hr = jnp.maximum(acc, 0.0)
    hr_ref[...] = hr
    h16_ref[...] = hr.astype(jnp.bfloat16)


def _encode(xc_bf, ae_bf):
    return pl.pallas_call(
        _encode_body,
        grid=(WIDTH // WB1, NTOK // TB1),
        in_specs=[
            pl.BlockSpec((TB1, DIMIN), lambda w, t: (t, 0)),
            pl.BlockSpec((WB1, DIMIN), lambda w, t: (w, 0)),
        ],
        out_specs=[
            pl.BlockSpec((TB1, WB1), lambda w, t: (t, w)),
            pl.BlockSpec((TB1, WB1), lambda w, t: (t, w)),
        ],
        out_shape=[
            jax.ShapeDtypeStruct((NTOK, WIDTH), jnp.float32),
            jax.ShapeDtypeStruct((NTOK, WIDTH), jnp.bfloat16),
        ],
    )(xc_bf, ae_bf)


TB2 = 128


def _thresh_body(hr_ref, h16_ref, th_ref):
    hb16 = jax.lax.bitcast_convert_type(h16_ref[...], jnp.int16)

    def body_1(_, carry):
        lo, hi = carry
        mid = lo + ((hi - lo) >> 1)
        mid16 = mid.astype(jnp.int16)
        cnt = jnp.sum((hb16 >= mid16).astype(jnp.int32), axis=1,
                      keepdims=True)
        pred = cnt >= KVAL
        return jnp.where(pred, mid, lo), jnp.where(pred, hi, mid)

    lo0 = jnp.zeros((TB2, 1), jnp.int32)
    hi0 = jnp.full((TB2, 1), 0x7F81, jnp.int32)
    bstar, _ = jax.lax.fori_loop(0, 15, body_1, (lo0, hi0))

    hb = jax.lax.bitcast_convert_type(hr_ref[...], jnp.int32)

    def cond_2(carry):
        i, lo, hi, th, done = carry
        return (i < 18) & (jnp.min(done) == 0)

    def body_2(carry):
        i, lo, hi, th, done = carry
        mid = lo + ((hi - lo) >> 1)
        cnt = jnp.sum((hb >= mid).astype(jnp.int32), axis=1, keepdims=True)
        hit = jnp.logical_and(cnt == KVAL, done == 0)
        th = jnp.where(hit, mid, th)
        done = jnp.where(hit, 1, done)
        pred = cnt >= KVAL
        lo = jnp.where(pred, mid, lo)
        hi = jnp.where(pred, hi, mid)
        return i + 1, lo, hi, th, done

    i0 = jnp.int32(0)
    th0 = jnp.zeros((TB2, 1), jnp.int32)
    done0 = jnp.zeros((TB2, 1), jnp.int32)
    _, lo, _, th, done = jax.lax.while_loop(
        cond_2, body_2,
        (i0, (bstar - 1) << 16, (bstar + 2) << 16, th0, done0))
    th = jnp.where(done == 1, th, lo)
    t = jax.lax.bitcast_convert_type(th, jnp.float32)
    th_ref[...] = jnp.broadcast_to(t, (TB2, 128))


def _thresholds(hr, h16):
    return pl.pallas_call(
        _thresh_body,
        grid=(NTOK // TB2,),
        in_specs=[
            pl.BlockSpec((TB2, WIDTH), lambda t: (t, 0)),
            pl.BlockSpec((TB2, WIDTH), lambda t: (t, 0)),
        ],
        out_specs=pl.BlockSpec((TB2, 128), lambda t: (t, 0)),
        out_shape=jax.ShapeDtypeStruct((NTOK, 128), jnp.float32),
    )(hr, h16)


TB3 = 512
WB3 = 1024


def _decode_body(lam_ref, hr_ref, th_ref, ae_ref, out_ref):
    w = pl.program_id(0)
    t = pl.program_id(1)
    lam = lam_ref[0]
    val = hr_ref[...]
    m = val >= th_ref[:, :1]
    xint = jnp.where(m, val * lam, 0.0).astype(jnp.bfloat16)
    partial = jax.lax.dot_general(
        xint, ae_ref[...], (((1,), (0,)), ((), ())),
        preferred_element_type=jnp.float32)
    sl = pl.ds(t * TB3, TB3)

    @pl.when(w == 0)
    def _():
        out_ref[sl, :] = partial

    @pl.when(w > 0)
    def _():
        out_ref[sl, :] += partial


def _decode(lam, hr, th, ae_bf):
    return pl.pallas_call(
        _decode_body,
        grid=(WIDTH // WB3, NTOK // TB3),
        in_specs=[
            pl.BlockSpec(memory_space=pltpu.SMEM),
            pl.BlockSpec((TB3, WB3), lambda w, t: (t, w)),
            pl.BlockSpec((TB3, 128), lambda w, t: (t, 0)),
            pl.BlockSpec((WB3, DIMIN), lambda w, t: (w, 0)),
        ],
        out_specs=pl.BlockSpec((NTOK, DIMIN), lambda w, t: (0, 0)),
        out_shape=jax.ShapeDtypeStruct((NTOK, DIMIN), jnp.float32),
    )(lam, hr, th, ae_bf)


def kernel(x, Ae, Ad, be, bd, lambda_pre):
    lam = jax.nn.softplus(lambda_pre).reshape(1).astype(jnp.float32)
    xc_bf = (x - bd).astype(jnp.bfloat16)
    ae_bf = Ae.astype(jnp.bfloat16)
    hr, h16 = _encode(xc_bf, ae_bf)
    th = _thresholds(hr, h16)
    out = _decode(lam, hr, th, ae_bf)
    return out + bd


# speedup vs baseline: 1.3855x; 1.3855x over previous
import functools

import jax
import jax.numpy as jnp
from jax.experimental import pallas as pl
from jax.experimental.pallas import tpu as pltpu

NTOK = 4096
DIMIN = 2048
WIDTH = 16384
KVAL = 64


TB1 = 512
WB1 = 2048


def _encode_body(x_ref, ae_ref, hr_ref, mx_ref):
    acc = jax.lax.dot_general(
        x_ref[...], ae_ref[...], (((1,), (1,)), ((), ())),
        preferred_element_type=jnp.float32)
    hr = jnp.maximum(acc, 0.0)
    hr_ref[...] = hr
    mx_ref[...] = jnp.max(hr.reshape(TB1, 16, WB1 // 16), axis=1)


def _encode(xc_bf, ae_bf):
    return pl.pallas_call(
        _encode_body,
        grid=(WIDTH // WB1, NTOK // TB1),
        in_specs=[
            pl.BlockSpec((TB1, DIMIN), lambda w, t: (t, 0)),
            pl.BlockSpec((WB1, DIMIN), lambda w, t: (w, 0)),
        ],
        out_specs=[
            pl.BlockSpec((TB1, WB1), lambda w, t: (t, w)),
            pl.BlockSpec((TB1, WB1 // 16), lambda w, t: (t, w)),
        ],
        out_shape=[
            jax.ShapeDtypeStruct((NTOK, WIDTH), jnp.float32),
            jax.ShapeDtypeStruct((NTOK, WIDTH // 16), jnp.float32),
        ],
    )(xc_bf, ae_bf)


TB2 = 256


def _thresh_body(lam_ref, hr_ref, mx_ref, xi_ref):
    def body_a(_, carry):
        lo, hi = carry
        mid = lo + ((hi - lo) >> 1)
        mid_f = jax.lax.bitcast_convert_type(mid, jnp.float32)
        cnt = jnp.sum((mx_ref[...] >= mid_f).astype(jnp.int32), axis=1,
                      keepdims=True)
        pred = cnt >= KVAL
        return jnp.where(pred, mid, lo), jnp.where(pred, hi, mid)

    lo0 = jnp.zeros((TB2, 1), jnp.int32)
    rmax = jax.lax.bitcast_convert_type(
        jnp.max(mx_ref[...], axis=1, keepdims=True), jnp.int32)
    lo_m, _ = jax.lax.fori_loop(0, 31, body_a, (lo0, rmax + 1))

    def cond_b(carry):
        i, lo, hi, th, done = carry
        return (i < 31) & (jnp.min(done) == 0)

    def body_b(carry):
        i, lo, hi, th, done = carry
        mid = lo + ((hi - lo) >> 1)
        mid_f = jax.lax.bitcast_convert_type(mid, jnp.float32)
        cnt = jnp.sum((hr_ref[...] >= mid_f).astype(jnp.int32), axis=1,
                      keepdims=True)
        hit = jnp.logical_and(cnt == KVAL, done == 0)
        th = jnp.where(hit, mid, th)
        done = jnp.where(hit, 1, done)
        pred = cnt >= KVAL
        lo = jnp.where(pred, mid, lo)
        hi = jnp.where(pred, hi, mid)
        return i + 1, lo, hi, th, done

    i0 = jnp.int32(0)
    th0 = jnp.zeros((TB2, 1), jnp.int32)
    done0 = jnp.zeros((TB2, 1), jnp.int32)
    _, lo, _, th, done = jax.lax.while_loop(
        cond_b, body_b, (i0, lo_m, rmax + 1, th0, done0))
    th = jnp.where(done == 1, th, lo)
    t = jax.lax.bitcast_convert_type(th, jnp.float32)
    val = hr_ref[...]
    lam = lam_ref[0]
    xi_ref[...] = jnp.where(val >= t, val * lam, 0.0).astype(jnp.bfloat16)


def _thresholds(lam, hr, mx):
    return pl.pallas_call(
        _thresh_body,
        grid=(NTOK // TB2,),
        in_specs=[
            pl.BlockSpec(memory_space=pltpu.SMEM),
            pl.BlockSpec((TB2, WIDTH), lambda t: (t, 0)),
            pl.BlockSpec((TB2, WIDTH // 16), lambda t: (t, 0)),
        ],
        out_specs=pl.BlockSpec((TB2, WIDTH), lambda t: (t, 0)),
        out_shape=jax.ShapeDtypeStruct((NTOK, WIDTH), jnp.bfloat16),
    )(lam, hr, mx)


TB3 = 512
WB3 = 2048


def _decode_body(xi_ref, ae_ref, out_ref):
    w = pl.program_id(0)
    t = pl.program_id(1)
    partial = jax.lax.dot_general(
        xi_ref[...], ae_ref[...], (((1,), (0,)), ((), ())),
        preferred_element_type=jnp.float32)
    sl = pl.ds(t * TB3, TB3)

    @pl.when(w == 0)
    def _():
        out_ref[sl, :] = partial

    @pl.when(w > 0)
    def _():
        out_ref[sl, :] += partial


def _decode(xi, ae_bf):
    return pl.pallas_call(
        _decode_body,
        grid=(WIDTH // WB3, NTOK // TB3),
        in_specs=[
            pl.BlockSpec((TB3, WB3), lambda w, t: (t, w)),
            pl.BlockSpec((WB3, DIMIN), lambda w, t: (w, 0)),
        ],
        out_specs=pl.BlockSpec((NTOK, DIMIN), lambda w, t: (0, 0)),
        out_shape=jax.ShapeDtypeStruct((NTOK, DIMIN), jnp.float32),
    )(xi, ae_bf)


def kernel(x, Ae, Ad, be, bd, lambda_pre):
    lam = jax.nn.softplus(lambda_pre).reshape(1).astype(jnp.float32)
    xc_bf = (x - bd).astype(jnp.bfloat16)
    ae_bf = Ae.astype(jnp.bfloat16)
    hr, mx = _encode(xc_bf, ae_bf)
    xi = _thresholds(lam, hr, mx)
    out = _decode(xi, ae_bf)
    return out + bd


# speedup vs baseline: 1.4073x; 1.0157x over previous
import functools

import jax
import jax.numpy as jnp
from jax.experimental import pallas as pl
from jax.experimental.pallas import tpu as pltpu

NTOK = 4096
DIMIN = 2048
WIDTH = 16384
KVAL = 64


TB1 = 512
WB1 = 2048


def _encode_body(x_ref, ae_ref, hr_ref, mx_ref):
    acc = jax.lax.dot_general(
        x_ref[...], ae_ref[...], (((1,), (1,)), ((), ())),
        preferred_element_type=jnp.float32)
    hr = jnp.maximum(acc, 0.0)
    hr_ref[...] = hr
    mx_ref[...] = jnp.max(hr.reshape(TB1, 16, WB1 // 16), axis=1)


def _encode(xc_bf, ae_bf):
    return pl.pallas_call(
        _encode_body,
        grid=(WIDTH // WB1, NTOK // TB1),
        in_specs=[
            pl.BlockSpec((TB1, DIMIN), lambda w, t: (t, 0)),
            pl.BlockSpec((WB1, DIMIN), lambda w, t: (w, 0)),
        ],
        out_specs=[
            pl.BlockSpec((TB1, WB1), lambda w, t: (t, w)),
            pl.BlockSpec((TB1, WB1 // 16), lambda w, t: (t, w)),
        ],
        out_shape=[
            jax.ShapeDtypeStruct((NTOK, WIDTH), jnp.float32),
            jax.ShapeDtypeStruct((NTOK, WIDTH // 16), jnp.float32),
        ],
    )(xc_bf, ae_bf)


TB2 = 256


def _thresh_body(lam_ref, hr_ref, mx_ref, xi_ref):
    def body_a(_, carry):
        lo, hi = carry
        mid = lo + ((hi - lo) >> 1)
        mid_f = jax.lax.bitcast_convert_type(mid, jnp.float32)
        cnt = jnp.sum((mx_ref[...] >= mid_f).astype(jnp.int32), axis=1,
                      keepdims=True)
        pred = cnt >= KVAL
        return jnp.where(pred, mid, lo), jnp.where(pred, hi, mid)

    lo0 = jnp.zeros((TB2, 1), jnp.int32)
    rmax = jax.lax.bitcast_convert_type(
        jnp.max(mx_ref[...], axis=1, keepdims=True), jnp.int32)
    lo_m, _ = jax.lax.fori_loop(0, 31, body_a, (lo0, rmax + 1))

    def cond_b(carry):
        i, lo, hi, th, done = carry
        return (i < 31) & (jnp.min(done) == 0)

    def body_b(carry):
        i, lo, hi, th, done = carry
        mid = lo + ((hi - lo) >> 1)
        mid_f = jax.lax.bitcast_convert_type(mid, jnp.float32)
        cnt = jnp.sum((hr_ref[...] >= mid_f).astype(jnp.int32), axis=1,
                      keepdims=True)
        hit = jnp.logical_and(cnt == KVAL, done == 0)
        th = jnp.where(hit, mid, th)
        done = jnp.where(hit, 1, done)
        pred = cnt >= KVAL
        lo = jnp.where(pred, mid, lo)
        hi = jnp.where(pred, hi, mid)
        coll = jnp.logical_and(hi - lo <= 1, done == 0)
        th = jnp.where(coll, lo, th)
        done = jnp.where(coll, 1, done)
        return i + 1, lo, hi, th, done

    i0 = jnp.int32(0)
    th0 = jnp.zeros((TB2, 1), jnp.int32)
    done0 = jnp.zeros((TB2, 1), jnp.int32)
    _, lo, _, th, done = jax.lax.while_loop(
        cond_b, body_b, (i0, lo_m, rmax + 1, th0, done0))
    th = jnp.where(done == 1, th, lo)
    t = jax.lax.bitcast_convert_type(th, jnp.float32)
    val = hr_ref[...]
    lam = lam_ref[0]
    xi_ref[...] = jnp.where(val >= t, val * lam, 0.0).astype(jnp.bfloat16)


def _thresholds(lam, hr, mx):
    return pl.pallas_call(
        _thresh_body,
        grid=(NTOK // TB2,),
        in_specs=[
            pl.BlockSpec(memory_space=pltpu.SMEM),
            pl.BlockSpec((TB2, WIDTH), lambda t: (t, 0)),
            pl.BlockSpec((TB2, WIDTH // 16), lambda t: (t, 0)),
        ],
        out_specs=pl.BlockSpec((TB2, WIDTH), lambda t: (t, 0)),
        out_shape=jax.ShapeDtypeStruct((NTOK, WIDTH), jnp.bfloat16),
    )(lam, hr, mx)


TB3 = 512
WB3 = 2048


def _decode_body(xi_ref, ae_ref, out_ref):
    w = pl.program_id(0)
    t = pl.program_id(1)
    partial = jax.lax.dot_general(
        xi_ref[...], ae_ref[...], (((1,), (0,)), ((), ())),
        preferred_element_type=jnp.float32)
    sl = pl.ds(t * TB3, TB3)

    @pl.when(w == 0)
    def _():
        out_ref[sl, :] = partial

    @pl.when(w > 0)
    def _():
        out_ref[sl, :] += partial


def _decode(xi, ae_bf):
    return pl.pallas_call(
        _decode_body,
        grid=(WIDTH // WB3, NTOK // TB3),
        in_specs=[
            pl.BlockSpec((TB3, WB3), lambda w, t: (t, w)),
            pl.BlockSpec((WB3, DIMIN), lambda w, t: (w, 0)),
        ],
        out_specs=pl.BlockSpec((NTOK, DIMIN), lambda w, t: (0, 0)),
        out_shape=jax.ShapeDtypeStruct((NTOK, DIMIN), jnp.float32),
    )(xi, ae_bf)


def kernel(x, Ae, Ad, be, bd, lambda_pre):
    lam = jax.nn.softplus(lambda_pre).reshape(1).astype(jnp.float32)
    xc_bf = (x - bd).astype(jnp.bfloat16)
    ae_bf = Ae.astype(jnp.bfloat16)
    hr, mx = _encode(xc_bf, ae_bf)
    xi = _thresholds(lam, hr, mx)
    out = _decode(xi, ae_bf)
    return out + bd
